# baseline (speedup 1.0000x reference)
import functools

import jax
import jax.numpy as jnp
from jax import lax
from jax.experimental import pallas as pl
from jax.experimental.pallas import tpu as pltpu
from jax.experimental.pallas import tpu_sc as plsc

_NC = 2
_NS = 16
_NW = _NC * _NS
_CHUNK = 1000

_SC_PARAMS = pltpu.CompilerParams(use_tc_tiling_on_sc=False)


def _sc_gather(table, idx2):
    nch, c = idx2.shape
    h = table.shape[1]
    e = nch * c
    cpw = nch // _NW
    epw = e // _NW
    mesh = plsc.VectorSubcoreMesh(core_axis_name="c", subcore_axis_name="s")

    @functools.partial(
        pl.kernel,
        out_type=jax.ShapeDtypeStruct((e, h), jnp.float32),
        mesh=mesh,
        scratch_types=[
            pltpu.VMEM((cpw, c), jnp.int32),
            pltpu.VMEM((2, c, h), jnp.float32),
            pltpu.SemaphoreType.DMA,
            pltpu.SemaphoreType.DMA,
        ],
        compiler_params=_SC_PARAMS,
    )
    def k(table_hbm, idx_hbm, out_hbm, idx_v, rows_v, gsem, ssem):
        wid = lax.axis_index("s") * _NC + lax.axis_index("c")
        pltpu.sync_copy(idx_hbm.at[pl.ds(wid * cpw, cpw)], idx_v)
        ga = [None] * cpw
        st = [None] * cpw
        ga[0] = pltpu.async_copy(table_hbm.at[idx_v.at[0]], rows_v.at[0],
                                 gsem)
        for j in range(cpw):
            if j + 1 < cpw:
                if j >= 1:
                    st[j - 1].wait()
                ga[j + 1] = pltpu.async_copy(
                    table_hbm.at[idx_v.at[j + 1]], rows_v.at[(j + 1) % 2],
                    gsem)
            ga[j].wait()
            st[j] = pltpu.async_copy(
                rows_v.at[j % 2], out_hbm.at[pl.ds(wid * epw + j * c, c)],
                ssem)
        st[cpw - 2].wait()
        st[cpw - 1].wait()

    return k(table, idx2)


def _sc_scatter_add(vals, idx2, npad, with_deg):
    e, h = vals.shape
    nch, c = idx2.shape
    cpw = nch // _NW
    epw = e // _NW
    rpt = npad // _NS
    mesh = plsc.VectorSubcoreMesh(core_axis_name="c", subcore_axis_name="s")

    out_type = [jax.ShapeDtypeStruct((2 * npad, h), jnp.float32)]
    scratch = [
        pltpu.VMEM((cpw, c), jnp.int32),
        pltpu.VMEM((2, c, h), jnp.float32),
        pltpu.VMEM_SHARED((npad, h), jnp.float32),
        pltpu.SemaphoreType.DMA,
    ]
    if with_deg:
        out_type.append(jax.ShapeDtypeStruct((2 * npad, 16), jnp.float32))
        scratch.append(pltpu.VMEM((c, 16), jnp.float32))
        scratch.append(pltpu.VMEM_SHARED((npad, 16), jnp.float32))

    zeros_h = jnp.zeros((npad, h), jnp.float32)
    inputs = [vals, idx2, zeros_h]
    if with_deg:
        inputs.append(jnp.ones((c, 16), jnp.float32))
        inputs.append(jnp.zeros((npad, 16), jnp.float32))

    @functools.partial(
        pl.kernel,
        out_type=tuple(out_type),
        mesh=mesh,
        scratch_types=tuple(scratch),
        compiler_params=_SC_PARAMS,
    )
    def k(*refs):
        if with_deg:
            (vals_hbm, idx_hbm, zh_hbm, ones_hbm, z16_hbm,
             out_hbm, deg_hbm,
             idx_v, rows_v, acc_sh, lsem, ones_v, deg_sh) = refs
        else:
            (vals_hbm, idx_hbm, zh_hbm,
             out_hbm,
             idx_v, rows_v, acc_sh, lsem) = refs
        cid = lax.axis_index("c")
        sid = lax.axis_index("s")
        wid = sid * _NC + cid

        pltpu.sync_copy(idx_hbm.at[pl.ds(wid * cpw, cpw)], idx_v)
        pltpu.sync_copy(zh_hbm.at[pl.ds(sid * rpt, rpt)],
                        acc_sh.at[pl.ds(sid * rpt, rpt)])
        if with_deg:
            pltpu.sync_copy(z16_hbm.at[pl.ds(sid * rpt, rpt)],
                            deg_sh.at[pl.ds(sid * rpt, rpt)])
            pltpu.sync_copy(ones_hbm, ones_v)
        plsc.subcore_barrier()

        ld = [None] * cpw
        ld[0] = pltpu.async_copy(vals_hbm.at[pl.ds(wid * epw, c)],
                                 rows_v.at[0], lsem)
        for j in range(cpw):
            if j + 1 < cpw:
                ld[j + 1] = pltpu.async_copy(
                    vals_hbm.at[pl.ds(wid * epw + (j + 1) * c, c)],
                    rows_v.at[(j + 1) % 2], lsem)
            ld[j].wait()
            pltpu.sync_copy(rows_v.at[j % 2], acc_sh.at[idx_v.at[j]],
                            add=True)
            if with_deg:
                pltpu.sync_copy(ones_v, deg_sh.at[idx_v.at[j]], add=True)
        plsc.subcore_barrier()

        pltpu.sync_copy(acc_sh.at[pl.ds(sid * rpt, rpt)],
                        out_hbm.at[pl.ds(cid * npad + sid * rpt, rpt)])
        if with_deg:
            pltpu.sync_copy(deg_sh.at[pl.ds(sid * rpt, rpt)],
                            deg_hbm.at[pl.ds(cid * npad + sid * rpt, rpt)])

    return k(*inputs)


def _sc_gather_scatter(table, sidx2, didx2, npad):
    h = table.shape[1]
    nch, c = sidx2.shape
    cpw = nch // _NW
    rpt = npad // _NS
    mesh = plsc.VectorSubcoreMesh(core_axis_name="c", subcore_axis_name="s")

    @functools.partial(
        pl.kernel,
        out_type=jax.ShapeDtypeStruct((2 * npad, h), jnp.float32),
        mesh=mesh,
        scratch_types=[
            pltpu.VMEM((cpw, c), jnp.int32),
            pltpu.VMEM((cpw, c), jnp.int32),
            pltpu.VMEM((2, c, h), jnp.float32),
            pltpu.VMEM_SHARED((npad, h), jnp.float32),
            pltpu.SemaphoreType.DMA,
        ],
        compiler_params=_SC_PARAMS,
    )
    def k(table_hbm, sidx_hbm, didx_hbm, zh_hbm, out_hbm,
          sidx_v, didx_v, rows_v, acc_sh, gsem):
        cid = lax.axis_index("c")
        sid = lax.axis_index("s")
        wid = sid * _NC + cid

        pltpu.sync_copy(sidx_hbm.at[pl.ds(wid * cpw, cpw)], sidx_v)
        pltpu.sync_copy(didx_hbm.at[pl.ds(wid * cpw, cpw)], didx_v)
        pltpu.sync_copy(zh_hbm.at[pl.ds(sid * rpt, rpt)],
                        acc_sh.at[pl.ds(sid * rpt, rpt)])
        plsc.subcore_barrier()

        ga = [None] * cpw
        ga[0] = pltpu.async_copy(table_hbm.at[sidx_v.at[0]], rows_v.at[0],
                                 gsem)
        for j in range(cpw):
            if j + 1 < cpw:
                ga[j + 1] = pltpu.async_copy(
                    table_hbm.at[sidx_v.at[j + 1]], rows_v.at[(j + 1) % 2],
                    gsem)
            ga[j].wait()
            pltpu.sync_copy(rows_v.at[j % 2], acc_sh.at[didx_v.at[j]],
                            add=True)
        plsc.subcore_barrier()

        pltpu.sync_copy(acc_sh.at[pl.ds(sid * rpt, rpt)],
                        out_hbm.at[pl.ds(cid * npad + sid * rpt, rpt)])

    return k(table, sidx2, didx2, jnp.zeros((npad, h), jnp.float32))


def _tc_lin0(n_feat, w, b):
    n, _ = n_feat.shape
    h = w.shape[1]

    def body(x_ref, w_ref, b_ref, o_ref):
        x = jnp.dot(x_ref[...], w_ref[...], preferred_element_type=jnp.float32)
        o_ref[...] = jnp.maximum(x + b_ref[...], 0.0)

    return pl.pallas_call(
        body,
        out_shape=jax.ShapeDtypeStruct((n, h), jnp.float32),
    )(n_feat, w, b.reshape(1, h))


def _tc_msg(e_feat, g, en_w1, en_b1, en_g, en_bt, w2p, selb, b2r):
    e, d_e = e_feat.shape
    h = g.shape[1]
    eh = en_w1.shape[1]
    blk = 2000
    grid = e // blk

    def body(ef_ref, g_ref, w1_ref, b1_ref, gam_ref, bet_ref, w2_ref,
             sel_ref, b2_ref, o_ref):
        h1 = jnp.dot(ef_ref[...], w1_ref[...],
                     preferred_element_type=jnp.float32) + b1_ref[...]
        h1 = jnp.maximum(h1, 0.0)
        mu = jnp.mean(h1, axis=-1, keepdims=True)
        var = jnp.mean((h1 - mu) ** 2, axis=-1, keepdims=True)
        h1n = (h1 - mu) / jnp.sqrt(var + 1e-5) * gam_ref[...] + bet_ref[...]
        ew = jnp.dot(h1n.astype(jnp.bfloat16), w2_ref[...],
                     preferred_element_type=jnp.float32)
        gb = g_ref[...]
        p = ew * jnp.tile(gb, (1, h))
        o_ref[...] = (
            jnp.dot(p.astype(jnp.bfloat16), sel_ref[...],
                    preferred_element_type=jnp.float32)
            + jnp.dot(gb.astype(jnp.bfloat16), b2_ref[...],
                      preferred_element_type=jnp.float32))

    return pl.pallas_call(
        body,
        grid=(grid,),
        in_specs=[
            pl.BlockSpec((blk, d_e), lambda i: (i, 0)),
            pl.BlockSpec((blk, h), lambda i: (i, 0)),
            pl.BlockSpec((d_e, eh), lambda i: (0, 0)),
            pl.BlockSpec((1, eh), lambda i: (0, 0)),
            pl.BlockSpec((1, eh), lambda i: (0, 0)),
            pl.BlockSpec((1, eh), lambda i: (0, 0)),
            pl.BlockSpec((eh, h * h), lambda i: (0, 0)),
            pl.BlockSpec((h * h, h), lambda i: (0, 0)),
            pl.BlockSpec((h, h), lambda i: (0, 0)),
        ],
        out_specs=pl.BlockSpec((blk, h), lambda i: (i, 0)),
        out_shape=jax.ShapeDtypeStruct((e, h), jnp.float32),
    )(e_feat, g, en_w1, en_b1.reshape(1, eh), en_g.reshape(1, eh),
      en_bt.reshape(1, eh), w2p, selb, b2r)


def _tc_node_first(parts, degp, out, conv_bias, msg_w, msg_b, npad):
    n, h = out.shape

    def body(p_ref, d_ref, out_ref, cb_ref, wm_ref, wo_ref, b_ref,
             o_ref, dinv_ref):
        p = p_ref[0:n, :] + p_ref[npad:npad + n, :]
        deg = d_ref[0:n, 0:1] + d_ref[npad:npad + n, 0:1]
        dinv = 1.0 / jnp.maximum(deg, 1.0)
        dinv_ref[...] = dinv
        o = out_ref[...]
        m = jnp.maximum(p * dinv + o + cb_ref[...], 0.0)
        o_ref[...] = (jnp.dot(m, wm_ref[...], preferred_element_type=jnp.float32)
                      + jnp.dot(o, wo_ref[...], preferred_element_type=jnp.float32)
                      + b_ref[...])

    return pl.pallas_call(
        body,
        out_shape=(jax.ShapeDtypeStruct((n, h), jnp.float32),
                   jax.ShapeDtypeStruct((n, 1), jnp.float32)),
    )(parts, degp, out, conv_bias.reshape(1, h), msg_w[:h], msg_w[h:],
      msg_b.reshape(1, h))


def _tc_node_next(parts, dinv, out, conv_bias, msg_w, msg_b, npad):
    n, h = out.shape

    def body(p_ref, dinv_ref, out_ref, cb_ref, wm_ref, wo_ref, b_ref, o_ref):
        p = p_ref[0:n, :] + p_ref[npad:npad + n, :]
        o = out_ref[...]
        m = jnp.maximum(p * dinv_ref[...] + o + cb_ref[...], 0.0)
        o_ref[...] = (jnp.dot(m, wm_ref[...], preferred_element_type=jnp.float32)
                      + jnp.dot(o, wo_ref[...], preferred_element_type=jnp.float32)
                      + b_ref[...])

    return pl.pallas_call(
        body,
        out_shape=jax.ShapeDtypeStruct((n, h), jnp.float32),
    )(parts, dinv, out, conv_bias.reshape(1, h), msg_w[:h], msg_w[h:],
      msg_b.reshape(1, h))


def _tc_final(parts, dinv, out, init, sub_w, sub_b, npad):
    n, h = out.shape

    def body(p_ref, dinv_ref, out_ref, init_ref, wo_ref, wg_ref, b_ref,
             o_ref):
        p = p_ref[0:n, :] + p_ref[npad:npad + n, :]
        group = p * dinv_ref[...]
        o = out_ref[...]
        o_ref[...] = (jnp.dot(o, wo_ref[...], preferred_element_type=jnp.float32)
                      + jnp.dot(group, wg_ref[...], preferred_element_type=jnp.float32)
                      + b_ref[...] + init_ref[...])

    return pl.pallas_call(
        body,
        out_shape=jax.ShapeDtypeStruct((n, h), jnp.float32),
    )(parts, dinv, out, init, sub_w[:h], sub_w[h:], sub_b.reshape(1, h))


def kernel(n_feat, e_feat, edge_index, lin0_w, lin0_b, en_w1, en_b1, en_g,
           en_bt, en_w2, en_b2, conv_bias, msg_w, msg_b, sub_w, sub_b):
    n, _ = n_feat.shape
    h = lin0_w.shape[1]
    npad = ((n + _NS * 8 - 1) // (_NS * 8)) * (_NS * 8)
    e = e_feat.shape[0]

    src2 = edge_index[0].astype(jnp.int32).reshape(e // _CHUNK, _CHUNK)
    dst2 = edge_index[1].astype(jnp.int32).reshape(e // _CHUNK, _CHUNK)
    eh = en_w1.shape[1]
    w2p = (en_w2.reshape(eh, h, h).transpose(0, 2, 1).reshape(eh, h * h)
           .astype(jnp.bfloat16))
    selb = jnp.repeat(jnp.eye(h, dtype=jnp.float32), h,
                      axis=0).astype(jnp.bfloat16)
    b2r = en_b2.reshape(h, h).astype(jnp.bfloat16)

    out = _tc_lin0(n_feat, lin0_w, lin0_b)

    g = _sc_gather(out, src2)
    msg = _tc_msg(e_feat, g, en_w1, en_b1, en_g, en_bt, w2p, selb, b2r)
    parts, degp = _sc_scatter_add(msg, dst2, npad, with_deg=True)
    out, dinv = _tc_node_first(parts, degp, out, conv_bias, msg_w, msg_b,
                               npad)

    g = _sc_gather(out, src2)
    msg = _tc_msg(e_feat, g, en_w1, en_b1, en_g, en_bt, w2p, selb, b2r)
    parts = _sc_scatter_add(msg, dst2, npad, with_deg=False)[0]
    out = _tc_node_next(parts, dinv, out, conv_bias, msg_w, msg_b, npad)

    parts = _sc_gather_scatter(out, src2, dst2, npad)
    return _tc_final(parts, dinv, out, n_feat, sub_w, sub_b, npad)


# speedup vs baseline: 1.2586x; 1.2586x over previous
import functools

import jax
import jax.numpy as jnp
from jax import lax
from jax.experimental import pallas as pl
from jax.experimental.pallas import tpu as pltpu
from jax.experimental.pallas import tpu_sc as plsc

_NC = 2
_NS = 16
_NW = _NC * _NS
_CHUNK = 1000

_SC_PARAMS = pltpu.CompilerParams(use_tc_tiling_on_sc=False)


def _sc_gather(table, idx2):
    nch, c = idx2.shape
    h = table.shape[1]
    e = nch * c
    cpw = nch // _NW
    epw = e // _NW
    mesh = plsc.VectorSubcoreMesh(core_axis_name="c", subcore_axis_name="s")

    @functools.partial(
        pl.kernel,
        out_type=jax.ShapeDtypeStruct((e, h), jnp.float32),
        mesh=mesh,
        scratch_types=[
            pltpu.VMEM((cpw, c), jnp.int32),
            pltpu.VMEM((2, c, h), jnp.float32),
            pltpu.SemaphoreType.DMA,
            pltpu.SemaphoreType.DMA,
        ],
        compiler_params=_SC_PARAMS,
    )
    def k(table_hbm, idx_hbm, out_hbm, idx_v, rows_v, gsem, ssem):
        wid = lax.axis_index("s") * _NC + lax.axis_index("c")
        pltpu.sync_copy(idx_hbm.at[pl.ds(wid * cpw, cpw)], idx_v)
        ga = [None] * cpw
        st = [None] * cpw
        ga[0] = pltpu.async_copy(table_hbm.at[idx_v.at[0]], rows_v.at[0],
                                 gsem)
        for j in range(cpw):
            if j + 1 < cpw:
                if j >= 1:
                    st[j - 1].wait()
                ga[j + 1] = pltpu.async_copy(
                    table_hbm.at[idx_v.at[j + 1]], rows_v.at[(j + 1) % 2],
                    gsem)
            ga[j].wait()
            st[j] = pltpu.async_copy(
                rows_v.at[j % 2], out_hbm.at[pl.ds(wid * epw + j * c, c)],
                ssem)
        st[cpw - 2].wait()
        st[cpw - 1].wait()

    return k(table, idx2)


def _sc_scatter_add(vals, idx2, npad, with_deg):
    e, h = vals.shape
    nch, c = idx2.shape
    cpw = nch // _NW
    epw = e // _NW
    rpt = npad // _NS
    mesh = plsc.VectorSubcoreMesh(core_axis_name="c", subcore_axis_name="s")

    out_type = [jax.ShapeDtypeStruct((2 * npad, h), jnp.float32)]
    scratch = [
        pltpu.VMEM((cpw, c), jnp.int32),
        pltpu.VMEM((2, c, h), jnp.float32),
        pltpu.VMEM_SHARED((npad, h), jnp.float32),
        pltpu.SemaphoreType.DMA,
    ]
    if with_deg:
        out_type.append(jax.ShapeDtypeStruct((2 * npad, 16), jnp.float32))
        scratch.append(pltpu.VMEM((c, 16), jnp.float32))
        scratch.append(pltpu.VMEM_SHARED((npad, 16), jnp.float32))

    zeros_h = jnp.zeros((npad, h), jnp.float32)
    inputs = [vals, idx2, zeros_h]
    if with_deg:
        inputs.append(jnp.ones((c, 16), jnp.float32))
        inputs.append(jnp.zeros((npad, 16), jnp.float32))

    @functools.partial(
        pl.kernel,
        out_type=tuple(out_type),
        mesh=mesh,
        scratch_types=tuple(scratch),
        compiler_params=_SC_PARAMS,
    )
    def k(*refs):
        if with_deg:
            (vals_hbm, idx_hbm, zh_hbm, ones_hbm, z16_hbm,
             out_hbm, deg_hbm,
             idx_v, rows_v, acc_sh, lsem, ones_v, deg_sh) = refs
        else:
            (vals_hbm, idx_hbm, zh_hbm,
             out_hbm,
             idx_v, rows_v, acc_sh, lsem) = refs
        cid = lax.axis_index("c")
        sid = lax.axis_index("s")
        wid = sid * _NC + cid

        pltpu.sync_copy(idx_hbm.at[pl.ds(wid * cpw, cpw)], idx_v)
        pltpu.sync_copy(zh_hbm.at[pl.ds(sid * rpt, rpt)],
                        acc_sh.at[pl.ds(sid * rpt, rpt)])
        if with_deg:
            pltpu.sync_copy(z16_hbm.at[pl.ds(sid * rpt, rpt)],
                            deg_sh.at[pl.ds(sid * rpt, rpt)])
            pltpu.sync_copy(ones_hbm, ones_v)
        plsc.subcore_barrier()

        ld = [None] * cpw
        ld[0] = pltpu.async_copy(vals_hbm.at[pl.ds(wid * epw, c)],
                                 rows_v.at[0], lsem)
        for j in range(cpw):
            if j + 1 < cpw:
                ld[j + 1] = pltpu.async_copy(
                    vals_hbm.at[pl.ds(wid * epw + (j + 1) * c, c)],
                    rows_v.at[(j + 1) % 2], lsem)
            ld[j].wait()
            pltpu.sync_copy(rows_v.at[j % 2], acc_sh.at[idx_v.at[j]],
                            add=True)
            if with_deg:
                pltpu.sync_copy(ones_v, deg_sh.at[idx_v.at[j]], add=True)
        plsc.subcore_barrier()

        pltpu.sync_copy(acc_sh.at[pl.ds(sid * rpt, rpt)],
                        out_hbm.at[pl.ds(cid * npad + sid * rpt, rpt)])
        if with_deg:
            pltpu.sync_copy(deg_sh.at[pl.ds(sid * rpt, rpt)],
                            deg_hbm.at[pl.ds(cid * npad + sid * rpt, rpt)])

    return k(*inputs)


def _sc_gather_scatter(table, sidx2, didx2, npad):
    h = table.shape[1]
    nch, c = sidx2.shape
    cpw = nch // _NW
    rpt = npad // _NS
    mesh = plsc.VectorSubcoreMesh(core_axis_name="c", subcore_axis_name="s")

    @functools.partial(
        pl.kernel,
        out_type=jax.ShapeDtypeStruct((2 * npad, h), jnp.float32),
        mesh=mesh,
        scratch_types=[
            pltpu.VMEM((cpw, c), jnp.int32),
            pltpu.VMEM((cpw, c), jnp.int32),
            pltpu.VMEM((2, c, h), jnp.float32),
            pltpu.VMEM_SHARED((npad, h), jnp.float32),
            pltpu.SemaphoreType.DMA,
        ],
        compiler_params=_SC_PARAMS,
    )
    def k(table_hbm, sidx_hbm, didx_hbm, zh_hbm, out_hbm,
          sidx_v, didx_v, rows_v, acc_sh, gsem):
        cid = lax.axis_index("c")
        sid = lax.axis_index("s")
        wid = sid * _NC + cid

        pltpu.sync_copy(sidx_hbm.at[pl.ds(wid * cpw, cpw)], sidx_v)
        pltpu.sync_copy(didx_hbm.at[pl.ds(wid * cpw, cpw)], didx_v)
        pltpu.sync_copy(zh_hbm.at[pl.ds(sid * rpt, rpt)],
                        acc_sh.at[pl.ds(sid * rpt, rpt)])
        plsc.subcore_barrier()

        ga = [None] * cpw
        ga[0] = pltpu.async_copy(table_hbm.at[sidx_v.at[0]], rows_v.at[0],
                                 gsem)
        for j in range(cpw):
            if j + 1 < cpw:
                ga[j + 1] = pltpu.async_copy(
                    table_hbm.at[sidx_v.at[j + 1]], rows_v.at[(j + 1) % 2],
                    gsem)
            ga[j].wait()
            pltpu.sync_copy(rows_v.at[j % 2], acc_sh.at[didx_v.at[j]],
                            add=True)
        plsc.subcore_barrier()

        pltpu.sync_copy(acc_sh.at[pl.ds(sid * rpt, rpt)],
                        out_hbm.at[pl.ds(cid * npad + sid * rpt, rpt)])

    return k(table, sidx2, didx2, jnp.zeros((npad, h), jnp.float32))


def _tc_lin0(n_feat, w, b):
    n, _ = n_feat.shape
    h = w.shape[1]

    def body(x_ref, w_ref, b_ref, o_ref):
        x = jnp.dot(x_ref[...], w_ref[...], preferred_element_type=jnp.float32)
        o_ref[...] = jnp.maximum(x + b_ref[...], 0.0)

    return pl.pallas_call(
        body,
        out_shape=jax.ShapeDtypeStruct((n, h), jnp.float32),
    )(n_feat, w, b.reshape(1, h))


def _tc_msg(e_feat, g, en_w1, en_b1, en_g, en_bt, w2p, selb):
    e, d_e = e_feat.shape
    h = g.shape[1]
    eh = en_w1.shape[1]
    blk = 2000
    grid = e // blk

    def body(ef_ref, g_ref, w1_ref, b1_ref, gam_ref, bet_ref, w2_ref,
             sel_ref, o_ref):
        h1 = jnp.dot(ef_ref[...], w1_ref[...],
                     preferred_element_type=jnp.float32) + b1_ref[...]
        h1 = jnp.maximum(h1, 0.0)
        mu = jnp.mean(h1, axis=-1, keepdims=True)
        var = jnp.mean((h1 - mu) ** 2, axis=-1, keepdims=True)
        h1n = (h1 - mu) / jnp.sqrt(var + 1e-5) * gam_ref[...] + bet_ref[...]
        ew = jnp.dot(h1n.astype(jnp.bfloat16), w2_ref[...],
                     preferred_element_type=jnp.float32)
        gb = g_ref[...]
        p = ew * jnp.tile(gb, (1, h))
        p = jnp.concatenate([p, gb], axis=1)
        o_ref[...] = jnp.dot(p.astype(jnp.bfloat16), sel_ref[...],
                             preferred_element_type=jnp.float32)

    return pl.pallas_call(
        body,
        grid=(grid,),
        in_specs=[
            pl.BlockSpec((blk, d_e), lambda i: (i, 0)),
            pl.BlockSpec((blk, h), lambda i: (i, 0)),
            pl.BlockSpec((d_e, eh), lambda i: (0, 0)),
            pl.BlockSpec((1, eh), lambda i: (0, 0)),
            pl.BlockSpec((1, eh), lambda i: (0, 0)),
            pl.BlockSpec((1, eh), lambda i: (0, 0)),
            pl.BlockSpec((eh, h * h), lambda i: (0, 0)),
            pl.BlockSpec((h * h + h, h), lambda i: (0, 0)),
        ],
        out_specs=pl.BlockSpec((blk, h), lambda i: (i, 0)),
        out_shape=jax.ShapeDtypeStruct((e, h), jnp.float32),
    )(e_feat, g, en_w1, en_b1.reshape(1, eh), en_g.reshape(1, eh),
      en_bt.reshape(1, eh), w2p, selb)


def _tc_node_first(parts, degp, out, conv_bias, msg_w, msg_b, npad):
    n, h = out.shape

    def body(p_ref, d_ref, out_ref, cb_ref, wm_ref, wo_ref, b_ref,
             o_ref, dinv_ref):
        p = p_ref[0:n, :] + p_ref[npad:npad + n, :]
        deg = d_ref[0:n, 0:1] + d_ref[npad:npad + n, 0:1]
        dinv = 1.0 / jnp.maximum(deg, 1.0)
        dinv_ref[...] = dinv
        o = out_ref[...]
        m = jnp.maximum(p * dinv + o + cb_ref[...], 0.0)
        o_ref[...] = (jnp.dot(m, wm_ref[...], preferred_element_type=jnp.float32)
                      + jnp.dot(o, wo_ref[...], preferred_element_type=jnp.float32)
                      + b_ref[...])

    return pl.pallas_call(
        body,
        out_shape=(jax.ShapeDtypeStruct((n, h), jnp.float32),
                   jax.ShapeDtypeStruct((n, 1), jnp.float32)),
    )(parts, degp, out, conv_bias.reshape(1, h), msg_w[:h], msg_w[h:],
      msg_b.reshape(1, h))


def _tc_node_next(parts, dinv, out, conv_bias, msg_w, msg_b, npad):
    n, h = out.shape

    def body(p_ref, dinv_ref, out_ref, cb_ref, wm_ref, wo_ref, b_ref, o_ref):
        p = p_ref[0:n, :] + p_ref[npad:npad + n, :]
        o = out_ref[...]
        m = jnp.maximum(p * dinv_ref[...] + o + cb_ref[...], 0.0)
        o_ref[...] = (jnp.dot(m, wm_ref[...], preferred_element_type=jnp.float32)
                      + jnp.dot(o, wo_ref[...], preferred_element_type=jnp.float32)
                      + b_ref[...])

    return pl.pallas_call(
        body,
        out_shape=jax.ShapeDtypeStruct((n, h), jnp.float32),
    )(parts, dinv, out, conv_bias.reshape(1, h), msg_w[:h], msg_w[h:],
      msg_b.reshape(1, h))


def _tc_final(parts, dinv, out, init, sub_w, sub_b, npad):
    n, h = out.shape

    def body(p_ref, dinv_ref, out_ref, init_ref, wo_ref, wg_ref, b_ref,
             o_ref):
        p = p_ref[0:n, :] + p_ref[npad:npad + n, :]
        group = p * dinv_ref[...]
        o = out_ref[...]
        o_ref[...] = (jnp.dot(o, wo_ref[...], preferred_element_type=jnp.float32)
                      + jnp.dot(group, wg_ref[...], preferred_element_type=jnp.float32)
                      + b_ref[...] + init_ref[...])

    return pl.pallas_call(
        body,
        out_shape=jax.ShapeDtypeStruct((n, h), jnp.float32),
    )(parts, dinv, out, init, sub_w[:h], sub_w[h:], sub_b.reshape(1, h))


def kernel(n_feat, e_feat, edge_index, lin0_w, lin0_b, en_w1, en_b1, en_g,
           en_bt, en_w2, en_b2, conv_bias, msg_w, msg_b, sub_w, sub_b):
    n, _ = n_feat.shape
    h = lin0_w.shape[1]
    npad = ((n + _NS * 8 - 1) // (_NS * 8)) * (_NS * 8)
    e = e_feat.shape[0]

    src2 = edge_index[0].astype(jnp.int32).reshape(e // _CHUNK, _CHUNK)
    dst2 = edge_index[1].astype(jnp.int32).reshape(e // _CHUNK, _CHUNK)
    eh = en_w1.shape[1]
    w2p = (en_w2.reshape(eh, h, h).transpose(0, 2, 1).reshape(eh, h * h)
           .astype(jnp.bfloat16))
    selb = jnp.concatenate(
        [jnp.repeat(jnp.eye(h, dtype=jnp.float32), h, axis=0),
         en_b2.reshape(h, h)], axis=0).astype(jnp.bfloat16)

    out = _tc_lin0(n_feat, lin0_w, lin0_b)

    g = _sc_gather(out, src2)
    msg = _tc_msg(e_feat, g, en_w1, en_b1, en_g, en_bt, w2p, selb)
    parts, degp = _sc_scatter_add(msg, dst2, npad, with_deg=True)
    out, dinv = _tc_node_first(parts, degp, out, conv_bias, msg_w, msg_b,
                               npad)

    g = _sc_gather(out, src2)
    msg = _tc_msg(e_feat, g, en_w1, en_b1, en_g, en_bt, w2p, selb)
    parts = _sc_scatter_add(msg, dst2, npad, with_deg=False)[0]
    out = _tc_node_next(parts, dinv, out, conv_bias, msg_w, msg_b, npad)

    parts = _sc_gather_scatter(out, src2, dst2, npad)
    return _tc_final(parts, dinv, out, n_feat, sub_w, sub_b, npad)
